# speedup vs baseline: 1.8361x; 1.8361x over previous
import functools

import jax
import jax.numpy as jnp
from jax import lax
from jax.experimental import pallas as pl

N_NODES = 10000
C = 512
D = 1024
HID = 512
N_CLASSES = 460

BLK = 512
GRID_A = (N_NODES + BLK - 1) // BLK


def _dense_pre_body(emb_ref, w1_ref, b1_ref, w2_ref, b2_ref, wl_ref, bl_ref,
                    wr_ref, att_ref, xl_ref, xr_ref, wself_ref, sacc_ref):
    i = pl.program_id(0)
    emb = emb_ref[...]
    row = i * BLK + lax.broadcasted_iota(jnp.int32, (BLK, 1), 0)
    is_cent = row < C
    h1 = jnp.maximum(jnp.dot(emb, w1_ref[...],
                             preferred_element_type=jnp.float32) + b1_ref[...], 0.0)
    xn = jnp.dot(h1, w2_ref[...], preferred_element_type=jnp.float32) + b2_ref[...]
    x = jnp.where(is_cent, emb, xn)
    xl = jnp.dot(x, wl_ref[...], preferred_element_type=jnp.float32) + bl_ref[...]
    xr = jnp.dot(x, wr_ref[...], preferred_element_type=jnp.float32)
    z = xl + xr
    lz = jnp.maximum(z, 0.2 * z)
    alpha = jnp.sum(lz * att_ref[...], axis=1)
    w_self = jnp.exp(alpha)
    xl_ref[...] = xl
    xr_ref[...] = xr
    wself_ref[...] = w_self
    sacc_ref[...] = xl * w_self[:, None]


def _dense_pre(emb_x, W1, b1, W2, b2, Wl, bl, Wr, att):
    full = lambda s: pl.BlockSpec(s, lambda i: (0,) * len(s))
    return pl.pallas_call(
        _dense_pre_body,
        grid=(GRID_A,),
        in_specs=[
            pl.BlockSpec((BLK, D), lambda i: (i, 0)),
            full((D, HID)), full((HID,)), full((HID, D)), full((D,)),
            full((D, D)), full((D,)), full((D, D)), full((1, D)),
        ],
        out_specs=[
            pl.BlockSpec((BLK, D), lambda i: (i, 0)),
            pl.BlockSpec((BLK, D), lambda i: (i, 0)),
            pl.BlockSpec((BLK,), lambda i: (i,)),
            pl.BlockSpec((BLK, D), lambda i: (i, 0)),
        ],
        out_shape=[
            jax.ShapeDtypeStruct((N_NODES, D), jnp.float32),
            jax.ShapeDtypeStruct((N_NODES, D), jnp.float32),
            jax.ShapeDtypeStruct((N_NODES,), jnp.float32),
            jax.ShapeDtypeStruct((N_NODES, D), jnp.float32),
        ],
    )(emb_x, W1, b1, W2, b2, Wl, bl, Wr, att)


def _final_body(sacc_ref, eacc_ref, wself_ref, edenom_ref, bgat_ref, wf_ref,
                bf_ref, h_ref):
    denom = wself_ref[...] + jnp.sum(edenom_ref[...], axis=1)
    out = (sacc_ref[...] + eacc_ref[...]) / (denom + 1e-16)[:, None] + bgat_ref[...]
    h_ref[...] = jnp.dot(out, wf_ref[...],
                         preferred_element_type=jnp.float32) + bf_ref[...]


def _final(self_acc, edge_acc, w_self, edge_denom2d, bias_gat, Wf, bf):
    full = lambda s: pl.BlockSpec(s, lambda i: (0,) * len(s))
    return pl.pallas_call(
        _final_body,
        grid=(GRID_A,),
        in_specs=[
            pl.BlockSpec((BLK, D), lambda i: (i, 0)),
            pl.BlockSpec((BLK, D), lambda i: (i, 0)),
            pl.BlockSpec((BLK,), lambda i: (i,)),
            pl.BlockSpec((BLK, 16), lambda i: (i, 0)),
            full((D,)), full((D, N_CLASSES)), full((N_CLASSES,)),
        ],
        out_specs=pl.BlockSpec((BLK, N_CLASSES), lambda i: (i, 0)),
        out_shape=jax.ShapeDtypeStruct((N_NODES, N_CLASSES), jnp.float32),
    )(self_acc, edge_acc, w_self, edge_denom2d, bias_gat, Wf, bf)


def _edge_phase(xl, xr, att, src, dst):
    xj = xl[src]
    z = xr[dst] + xj
    alpha = jnp.sum(jnp.maximum(z, 0.2 * z) * att[0][None, :], axis=1)
    w = jnp.exp(alpha)
    denom = jax.ops.segment_sum(w, dst, num_segments=N_NODES)
    eacc = jax.ops.segment_sum(xj * w[:, None], dst, num_segments=N_NODES)
    edenom2d = jnp.concatenate(
        [denom[:, None], jnp.zeros((N_NODES, 15), jnp.float32)], axis=1)
    return eacc, edenom2d


def kernel(emb_x, edge_index, exps, exps_c, W1, b1, W2, b2, Wl, bl, Wr, att,
           bias_gat, Wf, bf):
    edge_index = edge_index.astype(jnp.int32)
    src = edge_index[:, 0]
    dst = edge_index[:, 1]
    xl, xr, w_self, self_acc = _dense_pre(emb_x, W1, b1, W2, b2, Wl, bl, Wr, att)
    edge_acc, edge_denom2d = _edge_phase(xl, xr, att, src, dst)
    h = _final(self_acc, edge_acc, w_self, edge_denom2d, bias_gat, Wf, bf)
    return (h, exps, exps_c)
